# baseline (speedup 1.0000x reference)
import functools

import jax
import jax.numpy as jnp
from jax import lax
from jax.experimental import pallas as pl
from jax.experimental.pallas import tpu as pltpu
from jax.experimental.pallas import tpu_sc as plsc

TOKENS = 16384
E = 64
K = 8
L = 16
NC = 2
NS = 16
NW = NC * NS
ROWS = TOKENS // NW

_mesh = plsc.VectorSubcoreMesh(core_axis_name="c", subcore_axis_name="s")


@functools.partial(
    pl.kernel,
    out_type=jax.ShapeDtypeStruct((NW, 3 * E), jnp.float32),
    mesh=_mesh,
    compiler_params=pltpu.CompilerParams(needs_layout_passes=False),
    scratch_types=[
        pltpu.VMEM((ROWS, E), jnp.float32),
        pltpu.VMEM((3 * E,), jnp.float32),
    ],
)
def _sc_topk_hist(l_hbm, w_hbm, out_hbm, buf_v, acc_v):
    c = lax.axis_index("c")
    s = lax.axis_index("s")
    wid = s * NC + c
    base = wid * ROWS

    iota = lax.iota(jnp.int32, L)
    zeros = jnp.zeros((L,), jnp.float32)
    ones = jnp.ones((L,), jnp.float32)
    top8_mask = iota < K
    idx_consts = [iota + L * j for j in range(E // L)]
    for j in range(3 * E // L):
        acc_v[pl.ds(L * j, L)] = zeros

    def merge(ka, va, kb, vb, descending):
        take_a = ka >= kb
        mk = jnp.maximum(ka, kb)
        mv = jnp.where(take_a, va, vb)
        return plsc.sort_key_val(mk, mv, descending=descending)

    def top8(r):
        ks, vs = [], []
        for j in range(E // L):
            k_s, v_s = plsc.sort_key_val(
                buf_v[r, pl.ds(L * j, L)], idx_consts[j],
                descending=(j % 2 == 0),
            )
            ks.append(k_s)
            vs.append(v_s)
        k01, v01 = merge(ks[0], vs[0], ks[1], vs[1], descending=True)
        k23, v23 = merge(ks[2], vs[2], ks[3], vs[3], descending=False)
        _, vf = merge(k01, v01, k23, v23, descending=True)
        return vf

    pltpu.sync_copy(l_hbm.at[pl.ds(base, ROWS)], buf_v)

    @plsc.parallel_loop(0, ROWS, unroll=8)
    def _(r):
        vf = top8(r)
        plsc.addupdate_scatter(acc_v, [vf], ones, mask=top8_mask)

    pltpu.sync_copy(w_hbm.at[pl.ds(base, ROWS)], buf_v)

    @plsc.parallel_loop(0, ROWS, unroll=8, carry=(zeros,) * (E // L))
    def sums(r, carry):
        vf = top8(r)
        plsc.addupdate_scatter(acc_v, [vf + E], ones, mask=top8_mask)
        return tuple(
            acc + buf_v[r, pl.ds(L * j, L)] for j, acc in enumerate(carry)
        )

    for j in range(E // L):
        acc_v[pl.ds(2 * E + L * j, L)] = sums[j]
    pltpu.sync_copy(acc_v, out_hbm.at[wid])


def _combine_body(p_ref, o_ref):
    s = jnp.sum(p_ref[...], axis=0, keepdims=True)
    o_ref[0:1, :] = s[:, 0:E]
    o_ref[1:2, :] = s[:, E:2 * E]
    o_ref[2:3, :] = s[:, 2 * E:3 * E]


def kernel(layer_idx, router_weights, num_experts_per_tok, router_logits):
    partials = _sc_topk_hist(router_logits, router_weights)
    out = pl.pallas_call(
        _combine_body,
        out_shape=jax.ShapeDtypeStruct((3, E), jnp.float32),
    )(partials)
    return out


# speedup vs baseline: 1.0011x; 1.0011x over previous
import functools

import jax
import jax.numpy as jnp
from jax import lax
from jax.experimental import pallas as pl
from jax.experimental.pallas import tpu as pltpu
from jax.experimental.pallas import tpu_sc as plsc

TOKENS = 16384
E = 64
K = 8
L = 16
NC = 2
NS = 16
NW = NC * NS
ROWS = TOKENS // NW

_mesh = plsc.VectorSubcoreMesh(core_axis_name="c", subcore_axis_name="s")


@functools.partial(
    pl.kernel,
    out_type=jax.ShapeDtypeStruct((NW, 3 * E), jnp.float32),
    mesh=_mesh,
    compiler_params=pltpu.CompilerParams(needs_layout_passes=False),
    scratch_types=[
        pltpu.VMEM((ROWS, E), jnp.float32),
        pltpu.VMEM((3 * E,), jnp.float32),
    ],
)
def _sc_topk_hist(l_hbm, w_hbm, out_hbm, buf_v, acc_v):
    c = lax.axis_index("c")
    s = lax.axis_index("s")
    wid = s * NC + c
    base = wid * ROWS

    iota = lax.iota(jnp.int32, L)
    zeros = jnp.zeros((L,), jnp.float32)
    ones = jnp.ones((L,), jnp.float32)
    top8_mask = iota < K
    idx_consts = [iota + L * j for j in range(E // L)]
    for j in range(3 * E // L):
        acc_v[pl.ds(L * j, L)] = zeros

    def merge(ka, va, kb, vb, descending):
        take_a = ka >= kb
        mk = jnp.maximum(ka, kb)
        mv = jnp.where(take_a, va, vb)
        return plsc.sort_key_val(mk, mv, descending=descending)

    def top8(r):
        ks, vs = [], []
        for j in range(E // L):
            k_s, v_s = plsc.sort_key_val(
                buf_v[r, pl.ds(L * j, L)], idx_consts[j],
                descending=(j % 2 == 0),
            )
            ks.append(k_s)
            vs.append(v_s)
        k01, v01 = merge(ks[0], vs[0], ks[1], vs[1], descending=True)
        k23, v23 = merge(ks[2], vs[2], ks[3], vs[3], descending=False)
        _, vf = merge(k01, v01, k23, v23, descending=True)
        return vf

    pltpu.sync_copy(l_hbm.at[pl.ds(base, ROWS)], buf_v)

    @plsc.parallel_loop(0, ROWS, unroll=4)
    def _(r):
        vf = top8(r)
        plsc.addupdate_scatter(acc_v, [vf], ones, mask=top8_mask)

    pltpu.sync_copy(w_hbm.at[pl.ds(base, ROWS)], buf_v)

    @plsc.parallel_loop(0, ROWS, unroll=4, carry=(zeros,) * (E // L))
    def sums(r, carry):
        vf = top8(r)
        plsc.addupdate_scatter(acc_v, [vf + E], ones, mask=top8_mask)
        return tuple(
            acc + buf_v[r, pl.ds(L * j, L)] for j, acc in enumerate(carry)
        )

    for j in range(E // L):
        acc_v[pl.ds(2 * E + L * j, L)] = sums[j]
    pltpu.sync_copy(acc_v, out_hbm.at[wid])


def _combine_body(p_ref, o_ref):
    s = jnp.sum(p_ref[...], axis=0, keepdims=True)
    o_ref[0:1, :] = s[:, 0:E]
    o_ref[1:2, :] = s[:, E:2 * E]
    o_ref[2:3, :] = s[:, 2 * E:3 * E]


def kernel(layer_idx, router_weights, num_experts_per_tok, router_logits):
    partials = _sc_topk_hist(router_logits, router_weights)
    out = pl.pallas_call(
        _combine_body,
        out_shape=jax.ShapeDtypeStruct((3, E), jnp.float32),
    )(partials)
    return out


# speedup vs baseline: 1.0456x; 1.0444x over previous
import functools

import jax
import jax.numpy as jnp
from jax import lax
from jax.experimental import pallas as pl
from jax.experimental.pallas import tpu as pltpu
from jax.experimental.pallas import tpu_sc as plsc

TOKENS = 16384
E = 64
K = 8
L = 16
NC = 2
NS = 16
NW = NC * NS
ROWS = TOKENS // NW

_mesh = plsc.VectorSubcoreMesh(core_axis_name="c", subcore_axis_name="s")


@functools.partial(
    pl.kernel,
    out_type=jax.ShapeDtypeStruct((NW, 3 * E), jnp.float32),
    mesh=_mesh,
    compiler_params=pltpu.CompilerParams(needs_layout_passes=False),
    scratch_types=[
        pltpu.VMEM((ROWS // 2, E), jnp.float32),
        pltpu.VMEM((ROWS // 2, E), jnp.float32),
        pltpu.VMEM((3 * E,), jnp.float32),
        pltpu.SemaphoreType.DMA,
        pltpu.SemaphoreType.DMA,
    ],
)
def _sc_topk_hist(l_hbm, w_hbm, out_hbm, buf_a, buf_b, acc_v, sem_a, sem_b):
    c = lax.axis_index("c")
    s = lax.axis_index("s")
    wid = s * NC + c
    base = wid * ROWS

    iota = lax.iota(jnp.int32, L)
    zeros = jnp.zeros((L,), jnp.float32)
    ones = jnp.ones((L,), jnp.float32)
    top8_mask = iota < K
    idx_consts = [iota + L * j for j in range(E // L)]
    for j in range(3 * E // L):
        acc_v[pl.ds(L * j, L)] = zeros

    def merge(ka, va, kb, vb, descending):
        take_a = ka >= kb
        mk = jnp.maximum(ka, kb)
        mv = jnp.where(take_a, va, vb)
        return plsc.sort_key_val(mk, mv, descending=descending)

    def top8(buf, r):
        ks, vs = [], []
        for j in range(E // L):
            k_s, v_s = plsc.sort_key_val(
                buf[r, pl.ds(L * j, L)], idx_consts[j],
                descending=(j % 2 == 0),
            )
            ks.append(k_s)
            vs.append(v_s)
        k01, v01 = merge(ks[0], vs[0], ks[1], vs[1], descending=True)
        k23, v23 = merge(ks[2], vs[2], ks[3], vs[3], descending=False)
        _, vf = merge(k01, v01, k23, v23, descending=True)
        return vf

    CH = ROWS // 2

    def loop_logits(buf):
        @plsc.parallel_loop(0, CH, unroll=4)
        def _(r):
            vf = top8(buf, r)
            plsc.addupdate_scatter(acc_v, [vf], ones, mask=top8_mask)

    def loop_weights(buf, carry0):
        @plsc.parallel_loop(0, CH, unroll=4, carry=carry0)
        def sums(r, carry):
            vf = top8(buf, r)
            plsc.addupdate_scatter(acc_v, [vf + E], ones, mask=top8_mask)
            return tuple(
                acc + buf[r, pl.ds(L * j, L)] for j, acc in enumerate(carry)
            )

        return sums

    h_a = pltpu.async_copy(l_hbm.at[pl.ds(base, CH)], buf_a, sem_a)
    h_b = pltpu.async_copy(l_hbm.at[pl.ds(base + CH, CH)], buf_b, sem_b)
    h_a.wait()
    loop_logits(buf_a)
    h_a2 = pltpu.async_copy(w_hbm.at[pl.ds(base, CH)], buf_a, sem_a)
    h_b.wait()
    loop_logits(buf_b)
    h_b2 = pltpu.async_copy(w_hbm.at[pl.ds(base + CH, CH)], buf_b, sem_b)
    h_a2.wait()
    sums = loop_weights(buf_a, (zeros,) * (E // L))
    h_b2.wait()
    sums = loop_weights(buf_b, sums)

    for j in range(E // L):
        acc_v[pl.ds(2 * E + L * j, L)] = sums[j]
    pltpu.sync_copy(acc_v, out_hbm.at[wid])


def _combine_body(p_ref, o_ref):
    s = jnp.sum(p_ref[...], axis=0, keepdims=True)
    o_ref[0:1, :] = s[:, 0:E]
    o_ref[1:2, :] = s[:, E:2 * E]
    o_ref[2:3, :] = s[:, 2 * E:3 * E]


def kernel(layer_idx, router_weights, num_experts_per_tok, router_logits):
    partials = _sc_topk_hist(router_logits, router_weights)
    out = pl.pallas_call(
        _combine_body,
        out_shape=jax.ShapeDtypeStruct((3, E), jnp.float32),
    )(partials)
    return out
